# baseline (speedup 1.0000x reference)
import functools

import jax
import jax.numpy as jnp
from jax import lax
from jax.experimental import pallas as pl
from jax.experimental.pallas import tpu as pltpu
from jax.experimental.pallas import tpu_sc as plsc

N = 10000
E = 320000
NG = 8
NP = 10240
NC, NS, L = 2, 16, 16
NW = NC * NS
NPB = 313
EW = E // NW
CAP = 2048


def _wid():
    return lax.axis_index("s") * NC + lax.axis_index("c")


def _mesh():
    return plsc.VectorSubcoreMesh(core_axis_name="c", subcore_axis_name="s")


@functools.partial(jax.jit, static_argnames=("ch",))
def _sc_gather_combine(U, V, src, dst, *, ch):
    G = 40

    def body(U_hbm, V_hbm, src_hbm, dst_hbm, out_hbm,
             idxS, idxD, bufU, bufV, bufO, sem):
        w = _wid()
        base = w * EW
        pltpu.sync_copy(src_hbm.at[pl.ds(base, EW)], idxS)
        pltpu.sync_copy(dst_hbm.at[pl.ds(base, EW)], idxD)

        def chunk(j, carry):
            off = j * G
            pltpu.async_copy(U_hbm.at[idxD.at[pl.ds(off, G)]], bufU, sem).wait()
            pltpu.async_copy(V_hbm.at[idxS.at[pl.ds(off, G)]], bufV, sem).wait()
            for r in range(G):
                for c in range(ch // L):
                    bufO[r, pl.ds(c * L, L)] = (
                        bufU[r, pl.ds(c * L, L)] + bufV[r, pl.ds(c * L, L)])
            pltpu.sync_copy(bufO, out_hbm.at[pl.ds(base + off, G)])
            return carry

        lax.fori_loop(0, EW // G, chunk, 0)

    f = pl.kernel(
        body,
        out_type=jax.ShapeDtypeStruct((E, ch), jnp.float32),
        mesh=_mesh(),
        compiler_params=pltpu.CompilerParams(use_tc_tiling_on_sc=False, needs_layout_passes=False),
        scratch_types=[
            pltpu.VMEM((EW,), jnp.int32),
            pltpu.VMEM((EW,), jnp.int32),
            pltpu.VMEM((G, ch), jnp.float32),
            pltpu.VMEM((G, ch), jnp.float32),
            pltpu.VMEM((G, ch), jnp.float32),
            pltpu.SemaphoreType.DMA,
        ],
    )
    return f(U, V, src, dst)


@functools.partial(jax.jit, static_argnames=("ch",))
def _sc_scatter_max(h2, dst, *, ch):
    DBLK = 3200
    NBLK = E // DBLK
    VPB = DBLK // L
    TROWS = NPB + 1
    CW = ch // L

    def body(h2_hbm, dst_hbm, out_hbm, dstbuf, idbuf, rowbuf, tab, gbuf, sem):
        lane = lax.iota(jnp.int32, L)
        w = _wid()
        lo = w * NPB

        def zero(i, c):
            tab[pl.ds(i * L, L)] = jnp.zeros((L,), jnp.float32)
            return c
        lax.fori_loop(0, TROWS * ch // L, zero, 0)

        @pl.when(w == 0)
        def _():
            pltpu.sync_copy(tab.at[pl.ds(0, (NP - NW * NPB) * ch)],
                            out_hbm.at[pl.ds(NW * NPB * ch, (NP - NW * NPB) * ch)])

        def drain(pos, n16):
            def one(k, c):
                pltpu.async_copy(h2_hbm.at[idbuf.at[pl.ds(k * L, L)]],
                                 gbuf, sem).wait()
                rows = rowbuf[pl.ds(k * L, L)]
                for r in range(L):
                    lrow = lax.reduce_max(
                        jnp.where(lane == r, rows, 0), axes=(0,))
                    for c in range(CW):
                        o = lrow * ch + c * L
                        tab[pl.ds(o, L)] = jnp.maximum(
                            tab[pl.ds(o, L)], gbuf[r, pl.ds(c * L, L)])
                return c
            lax.fori_loop(0, n16, one, 0)
            return pos

        def scan_block(jb, pos):
            pltpu.sync_copy(dst_hbm.at[pl.ds(jb * DBLK, DBLK)], dstbuf)

            def vstep(i, pos):
                d = dstbuf[pl.ds(i * L, L)]
                gid = jb * DBLK + i * L + lane
                m = (d >= lo) & (d < lo + NPB)
                mi = jnp.where(m, 1, 0).astype(jnp.int32)
                incl = jnp.cumsum(mi)
                slot = pos + incl - mi
                plsc.store_scatter(idbuf, [slot], gid, mask=m)
                plsc.store_scatter(rowbuf, [slot], d - lo, mask=m)
                pos = pos + lax.reduce_max(incl, axes=(0,))

                def do_drain(p):
                    n16 = p // L
                    drain(p, n16)
                    fl = n16 * L
                    tid = idbuf[pl.ds(fl, L)]
                    trw = rowbuf[pl.ds(fl, L)]
                    idbuf[pl.ds(0, L)] = tid
                    rowbuf[pl.ds(0, L)] = trw
                    return p - fl

                return lax.cond(pos >= CAP, do_drain, lambda p: p, pos)

            return lax.fori_loop(0, VPB, vstep, pos)

        pos = lax.fori_loop(0, NBLK, scan_block, jnp.int32(0))
        idbuf[pl.ds(pos, L)] = jnp.zeros((L,), jnp.int32)
        rowbuf[pl.ds(pos, L)] = jnp.full((L,), NPB, jnp.int32)
        drain(pos, (pos + L - 1) // L)
        pltpu.sync_copy(tab.at[pl.ds(0, NPB * ch)],
                        out_hbm.at[pl.ds(lo * ch, NPB * ch)])

    f = pl.kernel(
        body,
        out_type=jax.ShapeDtypeStruct((NP * ch,), jnp.float32),
        mesh=_mesh(),
        compiler_params=pltpu.CompilerParams(use_tc_tiling_on_sc=False, needs_layout_passes=False),
        scratch_types=[
            pltpu.VMEM((DBLK,), jnp.int32),
            pltpu.VMEM((CAP + 32,), jnp.int32),
            pltpu.VMEM((CAP + 32,), jnp.int32),
            pltpu.VMEM((TROWS * ch,), jnp.float32),
            pltpu.VMEM((L, ch), jnp.float32),
            pltpu.SemaphoreType.DMA,
        ],
    )
    return f(h2, dst).reshape(NP, ch)


@functools.partial(jax.jit, static_argnames=("relu",))
def _tc_matmul(X, W, b, *, relu):
    M, K = X.shape
    Nc = W.shape[1]
    BM = 256
    BN = min(Nc, 512)

    def body(x_ref, w_ref, b_ref, o_ref):
        y = jnp.dot(x_ref[...], w_ref[...],
                    preferred_element_type=jnp.float32) + b_ref[...]
        if relu:
            y = jnp.maximum(y, 0.0)
        o_ref[...] = y

    return pl.pallas_call(
        body,
        grid=(M // BM, Nc // BN),
        in_specs=[
            pl.BlockSpec((BM, K), lambda i, j: (i, 0)),
            pl.BlockSpec((K, BN), lambda i, j: (0, j)),
            pl.BlockSpec((1, BN), lambda i, j: (0, j)),
        ],
        out_specs=pl.BlockSpec((BM, BN), lambda i, j: (i, j)),
        out_shape=jax.ShapeDtypeStruct((M, Nc), jnp.float32),
    )(X, W, b)


@jax.jit
def _tc_edge_mlp(h1pre, dpos, pw16, pb, wpf, w2, b2):
    ch = h1pre.shape[1]
    BM = 256

    def body(h_ref, d_ref, pw_ref, pb_ref, wpf_ref, w2_ref, b2_ref, o_ref):
        pf = jnp.maximum(
            jnp.dot(d_ref[...], pw_ref[...],
                    preferred_element_type=jnp.float32) + pb_ref[...], 0.0)
        h1 = jnp.maximum(
            h_ref[...] + jnp.dot(pf, wpf_ref[...],
                                 preferred_element_type=jnp.float32), 0.0)
        o_ref[...] = jnp.maximum(
            jnp.dot(h1, w2_ref[...],
                    preferred_element_type=jnp.float32) + b2_ref[...], 0.0)

    return pl.pallas_call(
        body,
        grid=(E // BM,),
        in_specs=[
            pl.BlockSpec((BM, ch), lambda i: (i, 0)),
            pl.BlockSpec((BM, 16), lambda i: (i, 0)),
            pl.BlockSpec((16, 16), lambda i: (0, 0)),
            pl.BlockSpec((1, 16), lambda i: (0, 0)),
            pl.BlockSpec((16, ch), lambda i: (0, 0)),
            pl.BlockSpec((ch, ch), lambda i: (0, 0)),
            pl.BlockSpec((1, ch), lambda i: (0, 0)),
        ],
        out_specs=pl.BlockSpec((BM, ch), lambda i: (i, 0)),
        out_shape=jax.ShapeDtypeStruct((E, ch), jnp.float32),
    )(h1pre, dpos, pw16, pb, wpf, w2, b2)


@jax.jit
def _tc_pool(x4, oh):
    BM = 256
    D = x4.shape[1]

    def body(x_ref, oh_ref, o_ref):
        @pl.when(pl.program_id(0) == 0)
        def _():
            o_ref[...] = jnp.zeros_like(o_ref)
        x = x_ref[...]
        for g in range(NG):
            cand = x * oh_ref[:, g:g + 1]
            o_ref[g:g + 1, :] = jnp.maximum(
                o_ref[g:g + 1, :], jnp.max(cand, axis=0, keepdims=True))

    return pl.pallas_call(
        body,
        grid=(NP // BM,),
        in_specs=[
            pl.BlockSpec((BM, D), lambda i: (i, 0)),
            pl.BlockSpec((BM, NG), lambda i: (i, 0)),
        ],
        out_specs=pl.BlockSpec((NG, D), lambda i: (0, 0)),
        out_shape=jax.ShapeDtypeStruct((NG, D), jnp.float32),
    )(x4, oh)


def _pad_rows(a, rows):
    return jnp.pad(a, ((0, rows - a.shape[0]), (0, 0)))


def _conv(x, ch, p, src, dst, dpos):
    cin = x.shape[1]
    (w1, b1), (w2, b2) = p["msg"]
    pw, pb = p["pos"]
    w_xi, w_dx, w_pf = w1[:cin], w1[cin:2 * cin], w1[2 * cin:]
    wuv = jnp.concatenate([w_xi - w_dx, w_dx], axis=1)
    buv = jnp.concatenate([b1, jnp.zeros_like(b1)])[None, :]
    uv = _tc_matmul(x, wuv, buv, relu=False)
    U, V = uv[:, :ch], uv[:, ch:]
    pw16 = jnp.pad(pw, ((0, 13), (0, 0)))
    h1pre = _sc_gather_combine(U, V, src, dst, ch=ch)
    h2 = _tc_edge_mlp(h1pre, dpos, pw16, pb[None, :], w_pf, w2, b2[None, :])
    return _sc_scatter_max(h2, dst, ch=ch)


def _gcu(x, cout, p, ei_t, ei_g, dpos_t, dpos_g):
    a = _conv(x, cout // 2, p["tpl"], ei_t[0], ei_t[1], dpos_t)
    b = _conv(x, cout // 2, p["geo"], ei_g[0], ei_g[1], dpos_g)
    wm, bm = p["mlp"][0]
    return _tc_matmul(jnp.concatenate([a, b], axis=1), wm, bm[None, :],
                      relu=True)


def kernel(pos, feature, tpl_edge_index, geo_edge_index, batch, params):
    posp = _pad_rows(jnp.pad(pos, ((0, 0), (0, 13))), NP)
    xf = _pad_rows(feature, NP)
    batchp = jnp.pad(batch, (0, NP - N), constant_values=NG)
    oh = (batchp[:, None] == jnp.arange(NG, dtype=jnp.int32)[None, :]
          ).astype(jnp.float32)

    dpos_t = _sc_gather_combine(-posp, posp, tpl_edge_index[0],
                                tpl_edge_index[1], ch=16)
    dpos_g = _sc_gather_combine(-posp, posp, geo_edge_index[0],
                                geo_edge_index[1], ch=16)

    x1 = _gcu(xf, 64, params["gcu1"], tpl_edge_index, geo_edge_index,
              dpos_t, dpos_g)
    x2 = _gcu(x1, 256, params["gcu2"], tpl_edge_index, geo_edge_index,
              dpos_t, dpos_g)
    x3 = _gcu(x2, 512, params["gcu3"], tpl_edge_index, geo_edge_index,
              dpos_t, dpos_g)

    wg, bg = params["glb"][0]
    x4 = _tc_matmul(jnp.concatenate([x1, x2, x3], axis=1), wg, bg[None, :],
                    relu=True)
    xg = _tc_pool(x4, oh)
    xgb = _tc_matmul(oh, xg, jnp.zeros((1, xg.shape[1]), jnp.float32),
                     relu=False)

    x5 = jnp.concatenate([xgb, posp[:, :3], xf, x1, x2, x3], axis=1)
    x5 = jnp.pad(x5, ((0, 0), (0, 2048 - x5.shape[1])))
    (wt1, bt1), (wt2, bt2) = params["trans"]
    wt1p = jnp.pad(wt1, ((0, 2048 - wt1.shape[0]), (0, 0)))
    t1 = _tc_matmul(x5, wt1p, bt1[None, :], relu=True)
    t2 = _tc_matmul(t1, wt2, bt2[None, :], relu=True)
    wo, bo = params["trans_out"]
    wop = jnp.pad(wo, ((0, 0), (0, 128 - wo.shape[1])))
    bop = jnp.pad(bo, (0, 128 - bo.shape[0]))
    o = _tc_matmul(t2, wop, bop[None, :], relu=False)
    return o[:N, :3]


# speedup vs baseline: 1.0007x; 1.0007x over previous
import functools

import jax
import jax.numpy as jnp
from jax import lax
from jax.experimental import pallas as pl
from jax.experimental.pallas import tpu as pltpu
from jax.experimental.pallas import tpu_sc as plsc

N = 10000
E = 320000
NG = 8
NP = 10240
NC, NS, L = 2, 16, 16
NW = NC * NS
NPB = 313
EW = E // NW
CAP = 2048


def _wid():
    return lax.axis_index("s") * NC + lax.axis_index("c")


def _mesh():
    return plsc.VectorSubcoreMesh(core_axis_name="c", subcore_axis_name="s")


@functools.partial(jax.jit, static_argnames=("cin",))
def _sc_gather_combine(x, posp, src, dst, *, cin):
    G = 40
    NCH = EW // G
    CW = cin // L

    def body(x_hbm, P_hbm, src_hbm, dst_hbm, xi_hbm, dx_hbm, dq_hbm,
             idxS, idxD,
             bufS0, bufS1, bufD0, bufD1, bufPS0, bufPS1, bufPD0, bufPD1,
             semG0, semG1, semO0, semO1):
        SB, DB = [bufS0, bufS1], [bufD0, bufD1]
        PS, PD = [bufPS0, bufPS1], [bufPD0, bufPD1]
        SG, SO = [semG0, semG1], [semO0, semO1]
        w = _wid()
        base = w * EW
        pltpu.sync_copy(src_hbm.at[pl.ds(base, EW)], idxS)
        pltpu.sync_copy(dst_hbm.at[pl.ds(base, EW)], idxD)

        def fire(j, b):
            @pl.when(j < NCH)
            def _():
                off = j * G
                pltpu.async_copy(x_hbm.at[idxS.at[pl.ds(off, G)]], SB[b], SG[b])
                pltpu.async_copy(x_hbm.at[idxD.at[pl.ds(off, G)]], DB[b], SG[b])
                pltpu.async_copy(P_hbm.at[idxS.at[pl.ds(off, G)]], PS[b], SG[b])
                pltpu.async_copy(P_hbm.at[idxD.at[pl.ds(off, G)]], PD[b], SG[b])

        def wait_g(b):
            for buf in (SB[b], DB[b]):
                pltpu.make_async_copy(x_hbm.at[pl.ds(0, G)], buf, SG[b]).wait()
            for buf in (PS[b], PD[b]):
                pltpu.make_async_copy(P_hbm.at[pl.ds(0, G)], buf, SG[b]).wait()

        def wait_o(b):
            pltpu.make_async_copy(xi_hbm.at[pl.ds(0, G)], DB[b], SO[b]).wait()
            pltpu.make_async_copy(dx_hbm.at[pl.ds(0, G)], SB[b], SO[b]).wait()
            pltpu.make_async_copy(dq_hbm.at[pl.ds(0, G)], PS[b], SO[b]).wait()

        fire(0, 0)

        def pair(jp, carry):
            for b in (0, 1):
                j = 2 * jp + b

                @pl.when(j > 0)
                def _():
                    wait_o(1 - b)
                fire(j + 1, 1 - b)
                wait_g(b)
                pltpu.async_copy(DB[b], xi_hbm.at[pl.ds(base + j * G, G)],
                                 SO[b])

                def col(c, cc):
                    for r in range(G):
                        SB[b][r, pl.ds(c * L, L)] = (
                            SB[b][r, pl.ds(c * L, L)]
                            - DB[b][r, pl.ds(c * L, L)])
                    return cc
                lax.fori_loop(0, CW, col, 0)
                for r in range(G):
                    PS[b][r, :] = PS[b][r, :] - PD[b][r, :]
                pltpu.async_copy(SB[b], dx_hbm.at[pl.ds(base + j * G, G)],
                                 SO[b])
                pltpu.async_copy(PS[b], dq_hbm.at[pl.ds(base + j * G, G)],
                                 SO[b])
            return carry

        lax.fori_loop(0, NCH // 2, pair, 0)
        wait_o(1)

    f = pl.kernel(
        body,
        out_type=(jax.ShapeDtypeStruct((E, cin), jnp.float32),
                  jax.ShapeDtypeStruct((E, cin), jnp.float32),
                  jax.ShapeDtypeStruct((E, 16), jnp.float32)),
        mesh=_mesh(),
        compiler_params=pltpu.CompilerParams(use_tc_tiling_on_sc=False, needs_layout_passes=False),
        scratch_types=(
            [pltpu.VMEM((EW,), jnp.int32)] * 2
            + [pltpu.VMEM((G, cin), jnp.float32)] * 4
            + [pltpu.VMEM((G, 16), jnp.float32)] * 4
            + [pltpu.SemaphoreType.DMA] * 4
        ),
    )
    return f(x, posp, src, dst)


@functools.partial(jax.jit, static_argnames=("ch",))
def _sc_scatter_max(h2, dst, *, ch):
    DBLK = 3200
    NBLK = E // DBLK
    VPB = DBLK // L
    TROWS = NPB + 1
    CW = ch // L

    def body(h2_hbm, dst_hbm, out_hbm, dstbuf, idbuf, rowbuf, tab, gbuf, sem):
        lane = lax.iota(jnp.int32, L)
        w = _wid()
        lo = w * NPB

        def zero(i, c):
            tab[pl.ds(i * L, L)] = jnp.zeros((L,), jnp.float32)
            return c
        lax.fori_loop(0, TROWS * ch // L, zero, 0)

        @pl.when(w == 0)
        def _():
            pltpu.sync_copy(tab.at[pl.ds(0, (NP - NW * NPB) * ch)],
                            out_hbm.at[pl.ds(NW * NPB * ch, (NP - NW * NPB) * ch)])

        def drain(pos, n16):
            def one(k, c):
                pltpu.async_copy(h2_hbm.at[idbuf.at[pl.ds(k * L, L)]],
                                 gbuf, sem).wait()
                rows = rowbuf[pl.ds(k * L, L)]
                for r in range(L):
                    lrow = lax.reduce_max(
                        jnp.where(lane == r, rows, 0), axes=(0,))
                    for c in range(CW):
                        o = lrow * ch + c * L
                        tab[pl.ds(o, L)] = jnp.maximum(
                            tab[pl.ds(o, L)], gbuf[r, pl.ds(c * L, L)])
                return c
            lax.fori_loop(0, n16, one, 0)
            return pos

        def scan_block(jb, pos):
            pltpu.sync_copy(dst_hbm.at[pl.ds(jb * DBLK, DBLK)], dstbuf)

            def vstep(i, pos):
                d = dstbuf[pl.ds(i * L, L)]
                gid = jb * DBLK + i * L + lane
                m = (d >= lo) & (d < lo + NPB)
                mi = jnp.where(m, 1, 0).astype(jnp.int32)
                incl = jnp.cumsum(mi)
                slot = pos + incl - mi
                plsc.store_scatter(idbuf, [slot], gid, mask=m)
                plsc.store_scatter(rowbuf, [slot], d - lo, mask=m)
                pos = pos + lax.reduce_max(incl, axes=(0,))

                def do_drain(p):
                    n16 = p // L
                    drain(p, n16)
                    fl = n16 * L
                    tid = idbuf[pl.ds(fl, L)]
                    trw = rowbuf[pl.ds(fl, L)]
                    idbuf[pl.ds(0, L)] = tid
                    rowbuf[pl.ds(0, L)] = trw
                    return p - fl

                return lax.cond(pos >= CAP, do_drain, lambda p: p, pos)

            return lax.fori_loop(0, VPB, vstep, pos)

        pos = lax.fori_loop(0, NBLK, scan_block, jnp.int32(0))
        idbuf[pl.ds(pos, L)] = jnp.zeros((L,), jnp.int32)
        rowbuf[pl.ds(pos, L)] = jnp.full((L,), NPB, jnp.int32)
        drain(pos, (pos + L - 1) // L)
        pltpu.sync_copy(tab.at[pl.ds(0, NPB * ch)],
                        out_hbm.at[pl.ds(lo * ch, NPB * ch)])

    f = pl.kernel(
        body,
        out_type=jax.ShapeDtypeStruct((NP * ch,), jnp.float32),
        mesh=_mesh(),
        compiler_params=pltpu.CompilerParams(use_tc_tiling_on_sc=False, needs_layout_passes=False),
        scratch_types=[
            pltpu.VMEM((DBLK,), jnp.int32),
            pltpu.VMEM((CAP + 32,), jnp.int32),
            pltpu.VMEM((CAP + 32,), jnp.int32),
            pltpu.VMEM((TROWS * ch,), jnp.float32),
            pltpu.VMEM((L, ch), jnp.float32),
            pltpu.SemaphoreType.DMA,
        ],
    )
    return f(h2, dst).reshape(NP, ch)


@functools.partial(jax.jit, static_argnames=("relu", "exact"))
def _tc_matmul(X, W, b, *, relu, exact=False):
    M, K = X.shape
    Nc = W.shape[1]
    BM = 256
    BN = min(Nc, 512)
    prec = jax.lax.Precision.HIGHEST if exact else None

    def body(x_ref, w_ref, b_ref, o_ref):
        y = jnp.dot(x_ref[...], w_ref[...],
                    preferred_element_type=jnp.float32, precision=prec) + b_ref[...]
        if relu:
            y = jnp.maximum(y, 0.0)
        o_ref[...] = y

    return pl.pallas_call(
        body,
        grid=(M // BM, Nc // BN),
        in_specs=[
            pl.BlockSpec((BM, K), lambda i, j: (i, 0)),
            pl.BlockSpec((K, BN), lambda i, j: (0, j)),
            pl.BlockSpec((1, BN), lambda i, j: (0, j)),
        ],
        out_specs=pl.BlockSpec((BM, BN), lambda i, j: (i, j)),
        out_shape=jax.ShapeDtypeStruct((M, Nc), jnp.float32),
    )(X, W, b)


@jax.jit
def _tc_edge_mlp(xi, dx, dpos, w1a, w1b, pw16, pb, wpf, w2, b1, b2):
    cin = xi.shape[1]
    ch = w2.shape[0]
    BM = 256

    def body(xi_ref, dx_ref, d_ref, w1a_ref, w1b_ref, pw_ref, pb_ref,
             wpf_ref, w2_ref, b1_ref, b2_ref, o_ref):
        dot = functools.partial(jnp.dot, preferred_element_type=jnp.float32)
        pf = jnp.maximum(dot(d_ref[...], pw_ref[...]) + pb_ref[...], 0.0)
        m1 = (dot(xi_ref[...], w1a_ref[...]) + dot(dx_ref[...], w1b_ref[...])
              + dot(pf, wpf_ref[...]) + b1_ref[...])
        h1 = jnp.maximum(m1, 0.0)
        o_ref[...] = jnp.maximum(
            dot(h1, w2_ref[...]) + b2_ref[...], 0.0)

    return pl.pallas_call(
        body,
        grid=(E // BM,),
        in_specs=[
            pl.BlockSpec((BM, cin), lambda i: (i, 0)),
            pl.BlockSpec((BM, cin), lambda i: (i, 0)),
            pl.BlockSpec((BM, 16), lambda i: (i, 0)),
            pl.BlockSpec((cin, ch), lambda i: (0, 0)),
            pl.BlockSpec((cin, ch), lambda i: (0, 0)),
            pl.BlockSpec((16, 16), lambda i: (0, 0)),
            pl.BlockSpec((1, 16), lambda i: (0, 0)),
            pl.BlockSpec((16, ch), lambda i: (0, 0)),
            pl.BlockSpec((ch, ch), lambda i: (0, 0)),
            pl.BlockSpec((1, ch), lambda i: (0, 0)),
            pl.BlockSpec((1, ch), lambda i: (0, 0)),
        ],
        out_specs=pl.BlockSpec((BM, ch), lambda i: (i, 0)),
        out_shape=jax.ShapeDtypeStruct((E, ch), jnp.float32),
    )(xi, dx, dpos, w1a, w1b, pw16, pb, wpf, w2, b1, b2)


@jax.jit
def _tc_pool(x4, oh):
    BM = 256
    D = x4.shape[1]

    def body(x_ref, oh_ref, o_ref):
        @pl.when(pl.program_id(0) == 0)
        def _():
            o_ref[...] = jnp.zeros_like(o_ref)
        x = x_ref[...]
        for g in range(NG):
            cand = x * oh_ref[:, g:g + 1]
            o_ref[g:g + 1, :] = jnp.maximum(
                o_ref[g:g + 1, :], jnp.max(cand, axis=0, keepdims=True))

    return pl.pallas_call(
        body,
        grid=(NP // BM,),
        in_specs=[
            pl.BlockSpec((BM, D), lambda i: (i, 0)),
            pl.BlockSpec((BM, NG), lambda i: (i, 0)),
        ],
        out_specs=pl.BlockSpec((NG, D), lambda i: (0, 0)),
        out_shape=jax.ShapeDtypeStruct((NG, D), jnp.float32),
    )(x4, oh)


def _pad_rows(a, rows):
    return jnp.pad(a, ((0, rows - a.shape[0]), (0, 0)))


def _conv(x, ch, p, gath, dst):
    cin = x.shape[1]
    (w1, b1), (w2, b2) = p["msg"]
    pw, pb = p["pos"]
    w_xi, w_dx, w_pf = w1[:cin], w1[cin:2 * cin], w1[2 * cin:]
    pw16 = jnp.pad(pw, ((0, 13), (0, 0)))
    xi, dx, dpos = gath
    h2 = _tc_edge_mlp(xi, dx, dpos, w_xi, w_dx, pw16, pb[None, :], w_pf,
                      w2, b1[None, :], b2[None, :])
    return _sc_scatter_max(h2, dst, ch=ch)


def _gcu(x, cout, p, ei_t, ei_g, posp):
    cin = x.shape[1]
    g_t = _sc_gather_combine(x, posp, ei_t[0], ei_t[1], cin=cin)
    g_g = _sc_gather_combine(x, posp, ei_g[0], ei_g[1], cin=cin)
    a = _conv(x, cout // 2, p["tpl"], g_t, ei_t[1])
    b = _conv(x, cout // 2, p["geo"], g_g, ei_g[1])
    wm, bm = p["mlp"][0]
    return _tc_matmul(jnp.concatenate([a, b], axis=1), wm, bm[None, :],
                      relu=True)


def kernel(pos, feature, tpl_edge_index, geo_edge_index, batch, params):
    posp = _pad_rows(jnp.pad(pos, ((0, 0), (0, 13))), NP)
    xf = _pad_rows(feature, NP)
    batchp = jnp.pad(batch, (0, NP - N), constant_values=NG)
    oh = (batchp[:, None] == jnp.arange(NG, dtype=jnp.int32)[None, :]
          ).astype(jnp.float32)

    x1 = _gcu(xf, 64, params["gcu1"], tpl_edge_index, geo_edge_index, posp)
    x2 = _gcu(x1, 256, params["gcu2"], tpl_edge_index, geo_edge_index, posp)
    x3 = _gcu(x2, 512, params["gcu3"], tpl_edge_index, geo_edge_index, posp)

    wg, bg = params["glb"][0]
    x4 = _tc_matmul(jnp.concatenate([x1, x2, x3], axis=1), wg, bg[None, :],
                    relu=True)
    xg = _tc_pool(x4, oh)
    xgb = _tc_matmul(oh, xg, jnp.zeros((1, xg.shape[1]), jnp.float32),
                     relu=False, exact=True)

    x5 = jnp.concatenate([xgb, posp[:, :3], xf, x1, x2, x3], axis=1)
    x5 = jnp.pad(x5, ((0, 0), (0, 2048 - x5.shape[1])))
    (wt1, bt1), (wt2, bt2) = params["trans"]
    wt1p = jnp.pad(wt1, ((0, 2048 - wt1.shape[0]), (0, 0)))
    t1 = _tc_matmul(x5, wt1p, bt1[None, :], relu=True)
    t2 = _tc_matmul(t1, wt2, bt2[None, :], relu=True)
    wo, bo = params["trans_out"]
    wop = jnp.pad(wo, ((0, 0), (0, 128 - wo.shape[1])))
    bop = jnp.pad(bo, (0, 128 - bo.shape[0]))
    o = _tc_matmul(t2, wop, bop[None, :], relu=False)
    return o[:N, :3]


# speedup vs baseline: 1.0280x; 1.0272x over previous
import functools

import jax
import jax.numpy as jnp
from jax import lax
from jax.experimental import pallas as pl
from jax.experimental.pallas import tpu as pltpu
from jax.experimental.pallas import tpu_sc as plsc

N = 10000
E = 320000
NG = 8
NP = 10240
NC, NS, L = 2, 16, 16
NW = NC * NS
NPB = 313
EW = E // NW
CAP = 2048


def _wid():
    return lax.axis_index("s") * NC + lax.axis_index("c")


def _mesh():
    return plsc.VectorSubcoreMesh(core_axis_name="c", subcore_axis_name="s")


@functools.partial(jax.jit, static_argnames=("cin",))
def _sc_gather_combine(x, posp, src, dst, *, cin):
    G = 40
    NCH = EW // G
    CW = cin // L

    def body(x_hbm, P_hbm, src_hbm, dst_hbm, xi_hbm, dx_hbm, dq_hbm,
             idxS, idxD,
             bufS0, bufS1, bufD0, bufD1, bufPS0, bufPS1, bufPD0, bufPD1,
             semG0, semG1, semO0, semO1):
        SB, DB = [bufS0, bufS1], [bufD0, bufD1]
        PS, PD = [bufPS0, bufPS1], [bufPD0, bufPD1]
        SG, SO = [semG0, semG1], [semO0, semO1]
        w = _wid()
        base = w * EW
        pltpu.sync_copy(src_hbm.at[pl.ds(base, EW)], idxS)
        pltpu.sync_copy(dst_hbm.at[pl.ds(base, EW)], idxD)

        def fire(j, b):
            @pl.when(j < NCH)
            def _():
                off = j * G
                pltpu.async_copy(x_hbm.at[idxS.at[pl.ds(off, G)]], SB[b], SG[b])
                pltpu.async_copy(x_hbm.at[idxD.at[pl.ds(off, G)]], DB[b], SG[b])
                pltpu.async_copy(P_hbm.at[idxS.at[pl.ds(off, G)]], PS[b], SG[b])
                pltpu.async_copy(P_hbm.at[idxD.at[pl.ds(off, G)]], PD[b], SG[b])

        def wait_g(b):
            for buf in (SB[b], DB[b]):
                pltpu.make_async_copy(x_hbm.at[pl.ds(0, G)], buf, SG[b]).wait()
            for buf in (PS[b], PD[b]):
                pltpu.make_async_copy(P_hbm.at[pl.ds(0, G)], buf, SG[b]).wait()

        def wait_o(b):
            pltpu.make_async_copy(xi_hbm.at[pl.ds(0, G)], DB[b], SO[b]).wait()
            pltpu.make_async_copy(dx_hbm.at[pl.ds(0, G)], SB[b], SO[b]).wait()
            pltpu.make_async_copy(dq_hbm.at[pl.ds(0, G)], PS[b], SO[b]).wait()

        fire(0, 0)

        def pair(jp, carry):
            for b in (0, 1):
                j = 2 * jp + b

                @pl.when(j > 0)
                def _():
                    wait_o(1 - b)
                fire(j + 1, 1 - b)
                wait_g(b)
                pltpu.async_copy(DB[b], xi_hbm.at[pl.ds(base + j * G, G)],
                                 SO[b])

                def col(c, cc):
                    for r in range(G):
                        SB[b][r, pl.ds(c * L, L)] = (
                            SB[b][r, pl.ds(c * L, L)]
                            - DB[b][r, pl.ds(c * L, L)])
                    return cc
                lax.fori_loop(0, CW, col, 0)
                for r in range(G):
                    PS[b][r, :] = PS[b][r, :] - PD[b][r, :]
                pltpu.async_copy(SB[b], dx_hbm.at[pl.ds(base + j * G, G)],
                                 SO[b])
                pltpu.async_copy(PS[b], dq_hbm.at[pl.ds(base + j * G, G)],
                                 SO[b])
            return carry

        lax.fori_loop(0, NCH // 2, pair, 0)
        wait_o(1)

    f = pl.kernel(
        body,
        out_type=(jax.ShapeDtypeStruct((E, cin), jnp.float32),
                  jax.ShapeDtypeStruct((E, cin), jnp.float32),
                  jax.ShapeDtypeStruct((E, 16), jnp.float32)),
        mesh=_mesh(),
        compiler_params=pltpu.CompilerParams(use_tc_tiling_on_sc=False, needs_layout_passes=False),
        scratch_types=(
            [pltpu.VMEM((EW,), jnp.int32)] * 2
            + [pltpu.VMEM((G, cin), jnp.float32)] * 4
            + [pltpu.VMEM((G, 16), jnp.float32)] * 4
            + [pltpu.SemaphoreType.DMA] * 4
        ),
    )
    return f(x, posp, src, dst)


@functools.partial(jax.jit, static_argnames=("ch",))
def _sc_scatter_max(h2, dst, *, ch):
    DBLK = 3200
    NBLK = E // DBLK
    VPB = DBLK // L
    TROWS = NPB + 1
    CW = ch // L

    def body(h2_hbm, dst_hbm, out_hbm, dstbuf, idbuf, rowbuf, tab,
             gbuf0, gbuf1, semD0, semD1):
        GB, SD = [gbuf0, gbuf1], [semD0, semD1]
        lane = lax.iota(jnp.int32, L)
        w = _wid()
        lo = w * NPB

        def zero(i, c):
            tab[pl.ds(i * L, L)] = jnp.zeros((L,), jnp.float32)
            return c
        lax.fori_loop(0, TROWS * ch // L, zero, 0)

        @pl.when(w == 0)
        def _():
            pltpu.sync_copy(tab.at[pl.ds(0, (NP - NW * NPB) * ch)],
                            out_hbm.at[pl.ds(NW * NPB * ch, (NP - NW * NPB) * ch)])

        def drain(pos, n16):
            def fire(k, b):
                @pl.when(k < n16)
                def _():
                    pltpu.async_copy(h2_hbm.at[idbuf.at[pl.ds(k * L, L)]],
                                     GB[b], SD[b])

            fire(0, 0)

            def one(k, c):
                for b in (0, 1):
                    @pl.when(k % 2 == b)
                    def _():
                        fire(k + 1, 1 - b)
                        pltpu.make_async_copy(
                            h2_hbm.at[pl.ds(0, L)], GB[b], SD[b]).wait()
                        rows = rowbuf[pl.ds(k * L, L)]
                        for r in range(L):
                            lrow = lax.reduce_max(
                                jnp.where(lane == r, rows, 0), axes=(0,))
                            for c2 in range(CW):
                                o = lrow * ch + c2 * L
                                tab[pl.ds(o, L)] = jnp.maximum(
                                    tab[pl.ds(o, L)],
                                    GB[b][r, pl.ds(c2 * L, L)])
                return c
            lax.fori_loop(0, n16, one, 0)
            return pos

        def scan_block(jb, pos):
            pltpu.sync_copy(dst_hbm.at[pl.ds(jb * DBLK, DBLK)], dstbuf)

            def vstep(i, pos):
                d = dstbuf[pl.ds(i * L, L)]
                gid = jb * DBLK + i * L + lane
                m = (d >= lo) & (d < lo + NPB)
                mi = jnp.where(m, 1, 0).astype(jnp.int32)
                incl = jnp.cumsum(mi)
                slot = pos + incl - mi
                plsc.store_scatter(idbuf, [slot], gid, mask=m)
                plsc.store_scatter(rowbuf, [slot], d - lo, mask=m)
                pos = pos + lax.reduce_max(incl, axes=(0,))

                def do_drain(p):
                    n16 = p // L
                    drain(p, n16)
                    fl = n16 * L
                    tid = idbuf[pl.ds(fl, L)]
                    trw = rowbuf[pl.ds(fl, L)]
                    idbuf[pl.ds(0, L)] = tid
                    rowbuf[pl.ds(0, L)] = trw
                    return p - fl

                return lax.cond(pos >= CAP, do_drain, lambda p: p, pos)

            return lax.fori_loop(0, VPB, vstep, pos)

        pos = lax.fori_loop(0, NBLK, scan_block, jnp.int32(0))
        idbuf[pl.ds(pos, L)] = jnp.zeros((L,), jnp.int32)
        rowbuf[pl.ds(pos, L)] = jnp.full((L,), NPB, jnp.int32)
        drain(pos, (pos + L - 1) // L)
        pltpu.sync_copy(tab.at[pl.ds(0, NPB * ch)],
                        out_hbm.at[pl.ds(lo * ch, NPB * ch)])

    f = pl.kernel(
        body,
        out_type=jax.ShapeDtypeStruct((NP * ch,), jnp.float32),
        mesh=_mesh(),
        compiler_params=pltpu.CompilerParams(use_tc_tiling_on_sc=False, needs_layout_passes=False),
        scratch_types=[
            pltpu.VMEM((DBLK,), jnp.int32),
            pltpu.VMEM((CAP + 32,), jnp.int32),
            pltpu.VMEM((CAP + 32,), jnp.int32),
            pltpu.VMEM((TROWS * ch,), jnp.float32),
            pltpu.VMEM((L, ch), jnp.float32),
            pltpu.VMEM((L, ch), jnp.float32),
            pltpu.SemaphoreType.DMA,
            pltpu.SemaphoreType.DMA,
        ],
    )
    return f(h2, dst).reshape(NP, ch)


@functools.partial(jax.jit, static_argnames=("relu", "exact"))
def _tc_matmul(X, W, b, *, relu, exact=False):
    M, K = X.shape
    Nc = W.shape[1]
    BM = 256
    BN = min(Nc, 512)
    prec = jax.lax.Precision.HIGHEST if exact else None

    def body(x_ref, w_ref, b_ref, o_ref):
        y = jnp.dot(x_ref[...], w_ref[...],
                    preferred_element_type=jnp.float32, precision=prec) + b_ref[...]
        if relu:
            y = jnp.maximum(y, 0.0)
        o_ref[...] = y

    return pl.pallas_call(
        body,
        grid=(M // BM, Nc // BN),
        in_specs=[
            pl.BlockSpec((BM, K), lambda i, j: (i, 0)),
            pl.BlockSpec((K, BN), lambda i, j: (0, j)),
            pl.BlockSpec((1, BN), lambda i, j: (0, j)),
        ],
        out_specs=pl.BlockSpec((BM, BN), lambda i, j: (i, j)),
        out_shape=jax.ShapeDtypeStruct((M, Nc), jnp.float32),
    )(X, W, b)


@jax.jit
def _tc_edge_mlp(xi, dx, dpos, w1a, w1b, pw16, pb, wpf, w2, b1, b2):
    cin = xi.shape[1]
    ch = w2.shape[0]
    BM = 256

    def body(xi_ref, dx_ref, d_ref, w1a_ref, w1b_ref, pw_ref, pb_ref,
             wpf_ref, w2_ref, b1_ref, b2_ref, o_ref):
        dot = functools.partial(jnp.dot, preferred_element_type=jnp.float32)
        pf = jnp.maximum(dot(d_ref[...], pw_ref[...]) + pb_ref[...], 0.0)
        m1 = (dot(xi_ref[...], w1a_ref[...]) + dot(dx_ref[...], w1b_ref[...])
              + dot(pf, wpf_ref[...]) + b1_ref[...])
        h1 = jnp.maximum(m1, 0.0)
        o_ref[...] = jnp.maximum(
            dot(h1, w2_ref[...]) + b2_ref[...], 0.0)

    return pl.pallas_call(
        body,
        grid=(E // BM,),
        in_specs=[
            pl.BlockSpec((BM, cin), lambda i: (i, 0)),
            pl.BlockSpec((BM, cin), lambda i: (i, 0)),
            pl.BlockSpec((BM, 16), lambda i: (i, 0)),
            pl.BlockSpec((cin, ch), lambda i: (0, 0)),
            pl.BlockSpec((cin, ch), lambda i: (0, 0)),
            pl.BlockSpec((16, 16), lambda i: (0, 0)),
            pl.BlockSpec((1, 16), lambda i: (0, 0)),
            pl.BlockSpec((16, ch), lambda i: (0, 0)),
            pl.BlockSpec((ch, ch), lambda i: (0, 0)),
            pl.BlockSpec((1, ch), lambda i: (0, 0)),
            pl.BlockSpec((1, ch), lambda i: (0, 0)),
        ],
        out_specs=pl.BlockSpec((BM, ch), lambda i: (i, 0)),
        out_shape=jax.ShapeDtypeStruct((E, ch), jnp.float32),
    )(xi, dx, dpos, w1a, w1b, pw16, pb, wpf, w2, b1, b2)


@jax.jit
def _tc_pool(x4, oh):
    BM = 256
    D = x4.shape[1]

    def body(x_ref, oh_ref, o_ref):
        @pl.when(pl.program_id(0) == 0)
        def _():
            o_ref[...] = jnp.zeros_like(o_ref)
        x = x_ref[...]
        for g in range(NG):
            cand = x * oh_ref[:, g:g + 1]
            o_ref[g:g + 1, :] = jnp.maximum(
                o_ref[g:g + 1, :], jnp.max(cand, axis=0, keepdims=True))

    return pl.pallas_call(
        body,
        grid=(NP // BM,),
        in_specs=[
            pl.BlockSpec((BM, D), lambda i: (i, 0)),
            pl.BlockSpec((BM, NG), lambda i: (i, 0)),
        ],
        out_specs=pl.BlockSpec((NG, D), lambda i: (0, 0)),
        out_shape=jax.ShapeDtypeStruct((NG, D), jnp.float32),
    )(x4, oh)


def _pad_rows(a, rows):
    return jnp.pad(a, ((0, rows - a.shape[0]), (0, 0)))


def _conv(x, ch, p, gath, dst):
    cin = x.shape[1]
    (w1, b1), (w2, b2) = p["msg"]
    pw, pb = p["pos"]
    w_xi, w_dx, w_pf = w1[:cin], w1[cin:2 * cin], w1[2 * cin:]
    pw16 = jnp.pad(pw, ((0, 13), (0, 0)))
    xi, dx, dpos = gath
    h2 = _tc_edge_mlp(xi, dx, dpos, w_xi, w_dx, pw16, pb[None, :], w_pf,
                      w2, b1[None, :], b2[None, :])
    return _sc_scatter_max(h2, dst, ch=ch)


def _gcu(x, cout, p, ei_t, ei_g, posp):
    cin = x.shape[1]
    g_t = _sc_gather_combine(x, posp, ei_t[0], ei_t[1], cin=cin)
    g_g = _sc_gather_combine(x, posp, ei_g[0], ei_g[1], cin=cin)
    a = _conv(x, cout // 2, p["tpl"], g_t, ei_t[1])
    b = _conv(x, cout // 2, p["geo"], g_g, ei_g[1])
    wm, bm = p["mlp"][0]
    return _tc_matmul(jnp.concatenate([a, b], axis=1), wm, bm[None, :],
                      relu=True)


def kernel(pos, feature, tpl_edge_index, geo_edge_index, batch, params):
    posp = _pad_rows(jnp.pad(pos, ((0, 0), (0, 13))), NP)
    xf = _pad_rows(feature, NP)
    batchp = jnp.pad(batch, (0, NP - N), constant_values=NG)
    oh = (batchp[:, None] == jnp.arange(NG, dtype=jnp.int32)[None, :]
          ).astype(jnp.float32)

    x1 = _gcu(xf, 64, params["gcu1"], tpl_edge_index, geo_edge_index, posp)
    x2 = _gcu(x1, 256, params["gcu2"], tpl_edge_index, geo_edge_index, posp)
    x3 = _gcu(x2, 512, params["gcu3"], tpl_edge_index, geo_edge_index, posp)

    wg, bg = params["glb"][0]
    x4 = _tc_matmul(jnp.concatenate([x1, x2, x3], axis=1), wg, bg[None, :],
                    relu=True)
    xg = _tc_pool(x4, oh)
    xgb = _tc_matmul(oh, xg, jnp.zeros((1, xg.shape[1]), jnp.float32),
                     relu=False, exact=True)

    x5 = jnp.concatenate([xgb, posp[:, :3], xf, x1, x2, x3], axis=1)
    x5 = jnp.pad(x5, ((0, 0), (0, 2048 - x5.shape[1])))
    (wt1, bt1), (wt2, bt2) = params["trans"]
    wt1p = jnp.pad(wt1, ((0, 2048 - wt1.shape[0]), (0, 0)))
    t1 = _tc_matmul(x5, wt1p, bt1[None, :], relu=True)
    t2 = _tc_matmul(t1, wt2, bt2[None, :], relu=True)
    wo, bo = params["trans_out"]
    wop = jnp.pad(wo, ((0, 0), (0, 128 - wo.shape[1])))
    bop = jnp.pad(bo, (0, 128 - bo.shape[0]))
    o = _tc_matmul(t2, wop, bop[None, :], relu=False)
    return o[:N, :3]


# speedup vs baseline: 1.2410x; 1.2073x over previous
import functools

import jax
import jax.numpy as jnp
from jax import lax
from jax.experimental import pallas as pl
from jax.experimental.pallas import tpu as pltpu
from jax.experimental.pallas import tpu_sc as plsc

N = 10000
E = 320000
NG = 8
NP = 10240
NC, NS, L = 2, 16, 16
NW = NC * NS
NPB = 313
EW = E // NW
CAP = 2048


def _wid():
    return lax.axis_index("s") * NC + lax.axis_index("c")


def _mesh():
    return plsc.VectorSubcoreMesh(core_axis_name="c", subcore_axis_name="s")


@functools.partial(jax.jit, static_argnames=("cin",))
def _sc_gather_combine(x, posp, src, dst, *, cin):
    G = 40
    NCH = EW // G
    CW = cin // L

    def body(x_hbm, P_hbm, src_hbm, dst_hbm, xi_hbm, dx_hbm, dq_hbm,
             idxS, idxD,
             bufS0, bufS1, bufD0, bufD1, bufPS0, bufPS1, bufPD0, bufPD1,
             semG0, semG1, semO0, semO1):
        SB, DB = [bufS0, bufS1], [bufD0, bufD1]
        PS, PD = [bufPS0, bufPS1], [bufPD0, bufPD1]
        SG, SO = [semG0, semG1], [semO0, semO1]
        w = _wid()
        base = w * EW
        pltpu.sync_copy(src_hbm.at[pl.ds(base, EW)], idxS)
        pltpu.sync_copy(dst_hbm.at[pl.ds(base, EW)], idxD)

        def fire(j, b):
            @pl.when(j < NCH)
            def _():
                off = j * G
                pltpu.async_copy(x_hbm.at[idxS.at[pl.ds(off, G)]], SB[b], SG[b])
                pltpu.async_copy(x_hbm.at[idxD.at[pl.ds(off, G)]], DB[b], SG[b])
                pltpu.async_copy(P_hbm.at[idxS.at[pl.ds(off, G)]], PS[b], SG[b])
                pltpu.async_copy(P_hbm.at[idxD.at[pl.ds(off, G)]], PD[b], SG[b])

        def wait_g(b):
            for buf in (SB[b], DB[b]):
                pltpu.make_async_copy(x_hbm.at[pl.ds(0, G)], buf, SG[b]).wait()
            for buf in (PS[b], PD[b]):
                pltpu.make_async_copy(P_hbm.at[pl.ds(0, G)], buf, SG[b]).wait()

        def wait_o(b):
            pltpu.make_async_copy(xi_hbm.at[pl.ds(0, G)], DB[b], SO[b]).wait()
            pltpu.make_async_copy(dx_hbm.at[pl.ds(0, G)], SB[b], SO[b]).wait()
            pltpu.make_async_copy(dq_hbm.at[pl.ds(0, G)], PS[b], SO[b]).wait()

        fire(0, 0)

        def pair(jp, carry):
            for b in (0, 1):
                j = 2 * jp + b

                @pl.when(j > 0)
                def _():
                    wait_o(1 - b)
                fire(j + 1, 1 - b)
                wait_g(b)
                pltpu.async_copy(DB[b], xi_hbm.at[pl.ds(base + j * G, G)],
                                 SO[b])

                def col(c, cc):
                    for r in range(G):
                        SB[b][r, pl.ds(c * L, L)] = (
                            SB[b][r, pl.ds(c * L, L)]
                            - DB[b][r, pl.ds(c * L, L)])
                    return cc
                lax.fori_loop(0, CW, col, 0)
                for r in range(G):
                    PS[b][r, :] = PS[b][r, :] - PD[b][r, :]
                pltpu.async_copy(SB[b], dx_hbm.at[pl.ds(base + j * G, G)],
                                 SO[b])
                pltpu.async_copy(PS[b], dq_hbm.at[pl.ds(base + j * G, G)],
                                 SO[b])
            return carry

        lax.fori_loop(0, NCH // 2, pair, 0)
        wait_o(1)

    f = pl.kernel(
        body,
        out_type=(jax.ShapeDtypeStruct((E, cin), jnp.float32),
                  jax.ShapeDtypeStruct((E, cin), jnp.float32),
                  jax.ShapeDtypeStruct((E, 16), jnp.float32)),
        mesh=_mesh(),
        compiler_params=pltpu.CompilerParams(use_tc_tiling_on_sc=False, needs_layout_passes=False),
        scratch_types=(
            [pltpu.VMEM((EW,), jnp.int32)] * 2
            + [pltpu.VMEM((G, cin), jnp.float32)] * 4
            + [pltpu.VMEM((G, 16), jnp.float32)] * 4
            + [pltpu.SemaphoreType.DMA] * 4
        ),
    )
    return f(x, posp, src, dst)


@jax.jit
def _sc_bucket_build(dst):
    DBLK = 3200
    NBLK = E // DBLK
    VPB = DBLK // L
    FB = 2048

    def body(dst_hbm, ids_hbm, rows_hbm, cnt_hbm,
             dstbuf, idbuf, rowbuf, cntv):
        lane = lax.iota(jnp.int32, L)
        w = _wid()
        lo = w * NPB

        def scan_block(jb, carry):
            pltpu.sync_copy(dst_hbm.at[pl.ds(jb * DBLK, DBLK)], dstbuf)

            def vstep(i, carry):
                pos, wr = carry
                d = dstbuf[pl.ds(i * L, L)]
                gid = jb * DBLK + i * L + lane
                m = (d >= lo) & (d < lo + NPB)
                mi = jnp.where(m, 1, 0).astype(jnp.int32)
                incl = jnp.cumsum(mi)
                slot = pos + incl - mi
                plsc.store_scatter(idbuf, [slot], gid, mask=m)
                plsc.store_scatter(rowbuf, [slot], d - lo, mask=m)
                pos = pos + lax.reduce_max(incl, axes=(0,))

                def flush(c):
                    p, wr = c
                    o8 = pl.multiple_of(w * E + wr, 8)
                    pltpu.sync_copy(idbuf.at[pl.ds(0, FB)],
                                    ids_hbm.at[pl.ds(o8, FB)])
                    pltpu.sync_copy(rowbuf.at[pl.ds(0, FB)],
                                    rows_hbm.at[pl.ds(o8, FB)])
                    tid = idbuf[pl.ds(FB, L)]
                    trw = rowbuf[pl.ds(FB, L)]
                    idbuf[pl.ds(0, L)] = tid
                    rowbuf[pl.ds(0, L)] = trw
                    return (p - FB, wr + FB)

                return lax.cond(pos >= FB, flush, lambda c: c, (pos, wr))

            return lax.fori_loop(0, VPB, vstep, carry)

        pos, wr = lax.fori_loop(0, NBLK, scan_block,
                                (jnp.int32(0), jnp.int32(0)))
        idbuf[pl.ds(pos, L)] = jnp.zeros((L,), jnp.int32)
        rowbuf[pl.ds(pos, L)] = jnp.full((L,), NPB, jnp.int32)
        pc = ((pos + L - 1) // L) * L

        def tail(j, c):
            o8 = pl.multiple_of(w * E + wr + 8 * j, 8)
            pltpu.sync_copy(idbuf.at[pl.ds(pl.multiple_of(8 * j, 8), 8)],
                            ids_hbm.at[pl.ds(o8, 8)])
            pltpu.sync_copy(rowbuf.at[pl.ds(pl.multiple_of(8 * j, 8), 8)],
                            rows_hbm.at[pl.ds(o8, 8)])
            return c
        lax.fori_loop(0, pc // 8, tail, 0)
        cntv[...] = jnp.zeros((L,), jnp.int32) + (wr + pc)
        pltpu.sync_copy(cntv, cnt_hbm.at[w])

    f = pl.kernel(
        body,
        out_type=(jax.ShapeDtypeStruct((NW * E + 2048,), jnp.int32),
                  jax.ShapeDtypeStruct((NW * E + 2048,), jnp.int32),
                  jax.ShapeDtypeStruct((NW, L), jnp.int32)),
        mesh=_mesh(),
        compiler_params=pltpu.CompilerParams(use_tc_tiling_on_sc=False, needs_layout_passes=False),
        scratch_types=[
            pltpu.VMEM((DBLK,), jnp.int32),
            pltpu.VMEM((FB + 32,), jnp.int32),
            pltpu.VMEM((FB + 32,), jnp.int32),
            pltpu.VMEM((L,), jnp.int32),
        ],
    )
    return f(dst)


@functools.partial(jax.jit, static_argnames=("ch",))
def _sc_scatter_max(h2, bkt, *, ch):
    TROWS = NPB + 1
    CW = ch // L
    FB = 2048

    def body(h2_hbm, ids_hbm, rows_hbm, cnt_hbm, out_hbm,
             idbuf, rowbuf, tab, gbuf0, gbuf1, cntv, semD0, semD1):
        GB, SD = [gbuf0, gbuf1], [semD0, semD1]
        lane = lax.iota(jnp.int32, L)
        w = _wid()
        lo = w * NPB

        def zero(i, c):
            tab[pl.ds(i * L, L)] = jnp.zeros((L,), jnp.float32)
            return c
        lax.fori_loop(0, TROWS * ch // L, zero, 0)

        @pl.when(w == 0)
        def _():
            pltpu.sync_copy(tab.at[pl.ds(0, (NP - NW * NPB) * ch)],
                            out_hbm.at[pl.ds(NW * NPB * ch, (NP - NW * NPB) * ch)])

        pltpu.sync_copy(cnt_hbm.at[w], cntv)
        cnt = lax.reduce_max(cntv[...], axes=(0,))
        n16 = cnt // L
        nblk = (n16 + (FB // L) - 1) // (FB // L)

        def block(jb, c):
            o8 = pl.multiple_of(w * E + jb * FB, 8)
            pltpu.sync_copy(ids_hbm.at[pl.ds(o8, FB)], idbuf)
            pltpu.sync_copy(rows_hbm.at[pl.ds(o8, FB)], rowbuf)
            ng = jnp.minimum(FB // L, n16 - jb * (FB // L))

            def fire(g, b):
                @pl.when(g < ng)
                def _():
                    pltpu.async_copy(h2_hbm.at[idbuf.at[pl.ds(g * L, L)]],
                                     GB[b], SD[b])

            fire(0, 0)

            def one(g, c2):
                for b in (0, 1):
                    @pl.when(g % 2 == b)
                    def _():
                        fire(g + 1, 1 - b)
                        pltpu.make_async_copy(
                            h2_hbm.at[pl.ds(0, L)], GB[b], SD[b]).wait()
                        rows = rowbuf[pl.ds(g * L, L)]
                        for r in range(L):
                            lrow = lax.reduce_max(
                                jnp.where(lane == r, rows, 0), axes=(0,))
                            for c3 in range(CW):
                                o = lrow * ch + c3 * L
                                tab[pl.ds(o, L)] = jnp.maximum(
                                    tab[pl.ds(o, L)],
                                    GB[b][r, pl.ds(c3 * L, L)])
                return c2
            lax.fori_loop(0, ng, one, 0)
            return c

        lax.fori_loop(0, nblk, block, 0)
        pltpu.sync_copy(tab.at[pl.ds(0, NPB * ch)],
                        out_hbm.at[pl.ds(lo * ch, NPB * ch)])

    f = pl.kernel(
        body,
        out_type=jax.ShapeDtypeStruct((NP * ch,), jnp.float32),
        mesh=_mesh(),
        compiler_params=pltpu.CompilerParams(use_tc_tiling_on_sc=False, needs_layout_passes=False),
        scratch_types=[
            pltpu.VMEM((FB,), jnp.int32),
            pltpu.VMEM((FB,), jnp.int32),
            pltpu.VMEM((TROWS * ch,), jnp.float32),
            pltpu.VMEM((L, ch), jnp.float32),
            pltpu.VMEM((L, ch), jnp.float32),
            pltpu.VMEM((L,), jnp.int32),
            pltpu.SemaphoreType.DMA,
            pltpu.SemaphoreType.DMA,
        ],
    )
    ids, rows, counts = bkt
    return f(h2, ids, rows, counts).reshape(NP, ch)


@functools.partial(jax.jit, static_argnames=("relu", "exact"))
def _tc_matmul(X, W, b, *, relu, exact=False):
    M, K = X.shape
    Nc = W.shape[1]
    BM = 256
    BN = min(Nc, 512)
    prec = jax.lax.Precision.HIGHEST if exact else None

    def body(x_ref, w_ref, b_ref, o_ref):
        y = jnp.dot(x_ref[...], w_ref[...],
                    preferred_element_type=jnp.float32, precision=prec) + b_ref[...]
        if relu:
            y = jnp.maximum(y, 0.0)
        o_ref[...] = y

    return pl.pallas_call(
        body,
        grid=(M // BM, Nc // BN),
        in_specs=[
            pl.BlockSpec((BM, K), lambda i, j: (i, 0)),
            pl.BlockSpec((K, BN), lambda i, j: (0, j)),
            pl.BlockSpec((1, BN), lambda i, j: (0, j)),
        ],
        out_specs=pl.BlockSpec((BM, BN), lambda i, j: (i, j)),
        out_shape=jax.ShapeDtypeStruct((M, Nc), jnp.float32),
    )(X, W, b)


@jax.jit
def _tc_edge_mlp(xi, dx, dpos, w1a, w1b, pw16, pb, wpf, w2, b1, b2):
    cin = xi.shape[1]
    ch = w2.shape[0]
    BM = 256

    def body(xi_ref, dx_ref, d_ref, w1a_ref, w1b_ref, pw_ref, pb_ref,
             wpf_ref, w2_ref, b1_ref, b2_ref, o_ref):
        dot = functools.partial(jnp.dot, preferred_element_type=jnp.float32)
        pf = jnp.maximum(dot(d_ref[...], pw_ref[...]) + pb_ref[...], 0.0)
        m1 = (dot(xi_ref[...], w1a_ref[...]) + dot(dx_ref[...], w1b_ref[...])
              + dot(pf, wpf_ref[...]) + b1_ref[...])
        h1 = jnp.maximum(m1, 0.0)
        o_ref[...] = jnp.maximum(
            dot(h1, w2_ref[...]) + b2_ref[...], 0.0)

    return pl.pallas_call(
        body,
        grid=(E // BM,),
        in_specs=[
            pl.BlockSpec((BM, cin), lambda i: (i, 0)),
            pl.BlockSpec((BM, cin), lambda i: (i, 0)),
            pl.BlockSpec((BM, 16), lambda i: (i, 0)),
            pl.BlockSpec((cin, ch), lambda i: (0, 0)),
            pl.BlockSpec((cin, ch), lambda i: (0, 0)),
            pl.BlockSpec((16, 16), lambda i: (0, 0)),
            pl.BlockSpec((1, 16), lambda i: (0, 0)),
            pl.BlockSpec((16, ch), lambda i: (0, 0)),
            pl.BlockSpec((ch, ch), lambda i: (0, 0)),
            pl.BlockSpec((1, ch), lambda i: (0, 0)),
            pl.BlockSpec((1, ch), lambda i: (0, 0)),
        ],
        out_specs=pl.BlockSpec((BM, ch), lambda i: (i, 0)),
        out_shape=jax.ShapeDtypeStruct((E, ch), jnp.float32),
    )(xi, dx, dpos, w1a, w1b, pw16, pb, wpf, w2, b1, b2)


@jax.jit
def _tc_pool(x4, oh):
    BM = 256
    D = x4.shape[1]

    def body(x_ref, oh_ref, o_ref):
        @pl.when(pl.program_id(0) == 0)
        def _():
            o_ref[...] = jnp.zeros_like(o_ref)
        x = x_ref[...]
        for g in range(NG):
            cand = x * oh_ref[:, g:g + 1]
            o_ref[g:g + 1, :] = jnp.maximum(
                o_ref[g:g + 1, :], jnp.max(cand, axis=0, keepdims=True))

    return pl.pallas_call(
        body,
        grid=(NP // BM,),
        in_specs=[
            pl.BlockSpec((BM, D), lambda i: (i, 0)),
            pl.BlockSpec((BM, NG), lambda i: (i, 0)),
        ],
        out_specs=pl.BlockSpec((NG, D), lambda i: (0, 0)),
        out_shape=jax.ShapeDtypeStruct((NG, D), jnp.float32),
    )(x4, oh)


def _pad_rows(a, rows):
    return jnp.pad(a, ((0, rows - a.shape[0]), (0, 0)))


def _conv(x, ch, p, gath, bkt):
    cin = x.shape[1]
    (w1, b1), (w2, b2) = p["msg"]
    pw, pb = p["pos"]
    w_xi, w_dx, w_pf = w1[:cin], w1[cin:2 * cin], w1[2 * cin:]
    pw16 = jnp.pad(pw, ((0, 13), (0, 0)))
    xi, dx, dpos = gath
    h2 = _tc_edge_mlp(xi, dx, dpos, w_xi, w_dx, pw16, pb[None, :], w_pf,
                      w2, b1[None, :], b2[None, :])
    return _sc_scatter_max(h2, bkt, ch=ch)


def _gcu(x, cout, p, ei_t, ei_g, posp, bkt_t, bkt_g):
    cin = x.shape[1]
    g_t = _sc_gather_combine(x, posp, ei_t[0], ei_t[1], cin=cin)
    g_g = _sc_gather_combine(x, posp, ei_g[0], ei_g[1], cin=cin)
    a = _conv(x, cout // 2, p["tpl"], g_t, bkt_t)
    b = _conv(x, cout // 2, p["geo"], g_g, bkt_g)
    wm, bm = p["mlp"][0]
    return _tc_matmul(jnp.concatenate([a, b], axis=1), wm, bm[None, :],
                      relu=True)


def kernel(pos, feature, tpl_edge_index, geo_edge_index, batch, params):
    posp = _pad_rows(jnp.pad(pos, ((0, 0), (0, 13))), NP)
    xf = _pad_rows(feature, NP)
    batchp = jnp.pad(batch, (0, NP - N), constant_values=NG)
    oh = (batchp[:, None] == jnp.arange(NG, dtype=jnp.int32)[None, :]
          ).astype(jnp.float32)

    bkt_t = _sc_bucket_build(tpl_edge_index[1])
    bkt_g = _sc_bucket_build(geo_edge_index[1])
    x1 = _gcu(xf, 64, params["gcu1"], tpl_edge_index, geo_edge_index, posp,
              bkt_t, bkt_g)
    x2 = _gcu(x1, 256, params["gcu2"], tpl_edge_index, geo_edge_index, posp,
              bkt_t, bkt_g)
    x3 = _gcu(x2, 512, params["gcu3"], tpl_edge_index, geo_edge_index, posp,
              bkt_t, bkt_g)

    wg, bg = params["glb"][0]
    x4 = _tc_matmul(jnp.concatenate([x1, x2, x3], axis=1), wg, bg[None, :],
                    relu=True)
    xg = _tc_pool(x4, oh)
    xgb = _tc_matmul(oh, xg, jnp.zeros((1, xg.shape[1]), jnp.float32),
                     relu=False, exact=True)

    x5 = jnp.concatenate([xgb, posp[:, :3], xf, x1, x2, x3], axis=1)
    x5 = jnp.pad(x5, ((0, 0), (0, 2048 - x5.shape[1])))
    (wt1, bt1), (wt2, bt2) = params["trans"]
    wt1p = jnp.pad(wt1, ((0, 2048 - wt1.shape[0]), (0, 0)))
    t1 = _tc_matmul(x5, wt1p, bt1[None, :], relu=True)
    t2 = _tc_matmul(t1, wt2, bt2[None, :], relu=True)
    wo, bo = params["trans_out"]
    wop = jnp.pad(wo, ((0, 0), (0, 128 - wo.shape[1])))
    bop = jnp.pad(bo, (0, 128 - bo.shape[0]))
    o = _tc_matmul(t2, wop, bop[None, :], relu=False)
    return o[:N, :3]


# speedup vs baseline: 1.2411x; 1.0001x over previous
import functools

import jax
import jax.numpy as jnp
from jax import lax
from jax.experimental import pallas as pl
from jax.experimental.pallas import tpu as pltpu
from jax.experimental.pallas import tpu_sc as plsc

N = 10000
E = 320000
NG = 8
NP = 10240
NC, NS, L = 2, 16, 16
NW = NC * NS
NPB = 313
EW = E // NW


def _wid():
    return lax.axis_index("s") * NC + lax.axis_index("c")


def _mesh():
    return plsc.VectorSubcoreMesh(core_axis_name="c", subcore_axis_name="s")


@functools.partial(jax.jit, static_argnames=("cin",))
def _sc_gather_combine(x, posp, src, dst, *, cin):
    G = 40
    NCH = EW // G
    CW = cin // L

    def body(x_hbm, P_hbm, src_hbm, dst_hbm, xi_hbm, dx_hbm, dq_hbm,
             idxS, idxD,
             bufS0, bufS1, bufD0, bufD1, bufPS0, bufPS1, bufPD0, bufPD1,
             semG0, semG1, semO0, semO1):
        SB, DB = [bufS0, bufS1], [bufD0, bufD1]
        PS, PD = [bufPS0, bufPS1], [bufPD0, bufPD1]
        SG, SO = [semG0, semG1], [semO0, semO1]
        w = _wid()
        base = w * EW
        pltpu.sync_copy(src_hbm.at[pl.ds(base, EW)], idxS)
        pltpu.sync_copy(dst_hbm.at[pl.ds(base, EW)], idxD)

        def fire(j, b):
            @pl.when(j < NCH)
            def _():
                off = j * G
                pltpu.async_copy(x_hbm.at[idxS.at[pl.ds(off, G)]], SB[b], SG[b])
                pltpu.async_copy(x_hbm.at[idxD.at[pl.ds(off, G)]], DB[b], SG[b])
                pltpu.async_copy(P_hbm.at[idxS.at[pl.ds(off, G)]], PS[b], SG[b])
                pltpu.async_copy(P_hbm.at[idxD.at[pl.ds(off, G)]], PD[b], SG[b])

        def wait_g(b):
            for buf in (SB[b], DB[b]):
                pltpu.make_async_copy(x_hbm.at[pl.ds(0, G)], buf, SG[b]).wait()
            for buf in (PS[b], PD[b]):
                pltpu.make_async_copy(P_hbm.at[pl.ds(0, G)], buf, SG[b]).wait()

        def wait_o(b):
            pltpu.make_async_copy(xi_hbm.at[pl.ds(0, G)], DB[b], SO[b]).wait()
            pltpu.make_async_copy(dx_hbm.at[pl.ds(0, G)], SB[b], SO[b]).wait()
            pltpu.make_async_copy(dq_hbm.at[pl.ds(0, G)], PS[b], SO[b]).wait()

        fire(0, 0)

        def pair(jp, carry):
            for b in (0, 1):
                j = 2 * jp + b

                @pl.when(j > 0)
                def _():
                    wait_o(1 - b)
                fire(j + 1, 1 - b)
                wait_g(b)
                pltpu.async_copy(DB[b], xi_hbm.at[pl.ds(base + j * G, G)],
                                 SO[b])

                def col(c, cc):
                    for r in range(G):
                        SB[b][r, pl.ds(c * L, L)] = (
                            SB[b][r, pl.ds(c * L, L)]
                            - DB[b][r, pl.ds(c * L, L)])
                    return cc
                lax.fori_loop(0, CW, col, 0)
                for r in range(G):
                    PS[b][r, :] = PS[b][r, :] - PD[b][r, :]
                pltpu.async_copy(SB[b], dx_hbm.at[pl.ds(base + j * G, G)],
                                 SO[b])
                pltpu.async_copy(PS[b], dq_hbm.at[pl.ds(base + j * G, G)],
                                 SO[b])
            return carry

        lax.fori_loop(0, NCH // 2, pair, 0)
        wait_o(1)

    f = pl.kernel(
        body,
        out_type=(jax.ShapeDtypeStruct((E, cin), jnp.float32),
                  jax.ShapeDtypeStruct((E, cin), jnp.float32),
                  jax.ShapeDtypeStruct((E, 16), jnp.float32)),
        mesh=_mesh(),
        compiler_params=pltpu.CompilerParams(use_tc_tiling_on_sc=False, needs_layout_passes=False),
        scratch_types=(
            [pltpu.VMEM((EW,), jnp.int32)] * 2
            + [pltpu.VMEM((G, cin), jnp.float32)] * 4
            + [pltpu.VMEM((G, 16), jnp.float32)] * 4
            + [pltpu.SemaphoreType.DMA] * 4
        ),
    )
    return f(x, posp, src, dst)


@jax.jit
def _sc_bucket_build(dst):
    DBLK = 3200
    NBLK = E // DBLK
    VPB = DBLK // L
    FB = 2048

    def body(dst_hbm, ids_hbm, rows_hbm, cnt_hbm,
             dstbuf, idbuf, rowbuf, cntv):
        lane = lax.iota(jnp.int32, L)
        w = _wid()
        lo = w * NPB

        def scan_block(jb, carry):
            pltpu.sync_copy(dst_hbm.at[pl.ds(jb * DBLK, DBLK)], dstbuf)

            def vstep(i, carry):
                pos, wr = carry
                d = dstbuf[pl.ds(i * L, L)]
                gid = jb * DBLK + i * L + lane
                m = (d >= lo) & (d < lo + NPB)
                mi = jnp.where(m, 1, 0).astype(jnp.int32)
                incl = jnp.cumsum(mi)
                slot = pos + incl - mi
                plsc.store_scatter(idbuf, [slot], gid, mask=m)
                plsc.store_scatter(rowbuf, [slot], d - lo, mask=m)
                pos = pos + lax.reduce_max(incl, axes=(0,))

                def flush(c):
                    p, wr = c
                    o8 = pl.multiple_of(w * E + wr, 8)
                    pltpu.sync_copy(idbuf.at[pl.ds(0, FB)],
                                    ids_hbm.at[pl.ds(o8, FB)])
                    pltpu.sync_copy(rowbuf.at[pl.ds(0, FB)],
                                    rows_hbm.at[pl.ds(o8, FB)])
                    tid = idbuf[pl.ds(FB, L)]
                    trw = rowbuf[pl.ds(FB, L)]
                    idbuf[pl.ds(0, L)] = tid
                    rowbuf[pl.ds(0, L)] = trw
                    return (p - FB, wr + FB)

                return lax.cond(pos >= FB, flush, lambda c: c, (pos, wr))

            return lax.fori_loop(0, VPB, vstep, carry)

        pos, wr = lax.fori_loop(0, NBLK, scan_block,
                                (jnp.int32(0), jnp.int32(0)))
        idbuf[pl.ds(pos, L)] = jnp.zeros((L,), jnp.int32)
        rowbuf[pl.ds(pos, L)] = jnp.full((L,), NPB, jnp.int32)
        pc = ((pos + L - 1) // L) * L

        def tail(j, c):
            o8 = pl.multiple_of(w * E + wr + 8 * j, 8)
            pltpu.sync_copy(idbuf.at[pl.ds(pl.multiple_of(8 * j, 8), 8)],
                            ids_hbm.at[pl.ds(o8, 8)])
            pltpu.sync_copy(rowbuf.at[pl.ds(pl.multiple_of(8 * j, 8), 8)],
                            rows_hbm.at[pl.ds(o8, 8)])
            return c
        lax.fori_loop(0, pc // 8, tail, 0)
        cntv[...] = jnp.zeros((L,), jnp.int32) + (wr + pc)
        pltpu.sync_copy(cntv, cnt_hbm.at[w])

    f = pl.kernel(
        body,
        out_type=(jax.ShapeDtypeStruct((NW * E + 2048,), jnp.int32),
                  jax.ShapeDtypeStruct((NW * E + 2048,), jnp.int32),
                  jax.ShapeDtypeStruct((NW, L), jnp.int32)),
        mesh=_mesh(),
        compiler_params=pltpu.CompilerParams(use_tc_tiling_on_sc=False, needs_layout_passes=False),
        scratch_types=[
            pltpu.VMEM((DBLK,), jnp.int32),
            pltpu.VMEM((FB + 32,), jnp.int32),
            pltpu.VMEM((FB + 32,), jnp.int32),
            pltpu.VMEM((L,), jnp.int32),
        ],
    )
    return f(dst)


@functools.partial(jax.jit, static_argnames=("ch",))
def _sc_scatter_max(h2, bkt, *, ch):
    TROWS = NPB + 1
    CW = ch // L
    FB = 2048

    def body(h2_hbm, ids_hbm, rows_hbm, cnt_hbm, out_hbm,
             idbuf, rowbuf, tab, gbuf0, gbuf1, cntv, semD0, semD1):
        GB, SD = [gbuf0, gbuf1], [semD0, semD1]
        lane = lax.iota(jnp.int32, L)
        w = _wid()
        lo = w * NPB

        def zero(i, c):
            tab[pl.ds(i * L, L)] = jnp.zeros((L,), jnp.float32)
            return c
        lax.fori_loop(0, TROWS * ch // L, zero, 0)

        @pl.when(w == 0)
        def _():
            pltpu.sync_copy(tab.at[pl.ds(0, (NP - NW * NPB) * ch)],
                            out_hbm.at[pl.ds(NW * NPB * ch, (NP - NW * NPB) * ch)])

        pltpu.sync_copy(cnt_hbm.at[w], cntv)
        cnt = lax.reduce_max(cntv[...], axes=(0,))
        n16 = cnt // L
        nblk = (n16 + (FB // L) - 1) // (FB // L)

        def block(jb, c):
            o8 = pl.multiple_of(w * E + jb * FB, 8)
            pltpu.sync_copy(ids_hbm.at[pl.ds(o8, FB)], idbuf)
            pltpu.sync_copy(rows_hbm.at[pl.ds(o8, FB)], rowbuf)
            ng = jnp.minimum(FB // L, n16 - jb * (FB // L))

            def fire(g, b):
                @pl.when(g < ng)
                def _():
                    pltpu.async_copy(h2_hbm.at[idbuf.at[pl.ds(g * L, L)]],
                                     GB[b], SD[b])

            fire(0, 0)

            def one(g, c2):
                for b in (0, 1):
                    @pl.when(g % 2 == b)
                    def _():
                        fire(g + 1, 1 - b)
                        pltpu.make_async_copy(
                            h2_hbm.at[pl.ds(0, L)], GB[b], SD[b]).wait()
                        rows = rowbuf[pl.ds(g * L, L)]
                        for r in range(L):
                            lrow = lax.reduce_max(
                                jnp.where(lane == r, rows, 0), axes=(0,))
                            for c3 in range(CW):
                                o = lrow * ch + c3 * L
                                tab[pl.ds(o, L)] = jnp.maximum(
                                    tab[pl.ds(o, L)],
                                    GB[b][r, pl.ds(c3 * L, L)])
                return c2
            lax.fori_loop(0, ng, one, 0)
            return c

        lax.fori_loop(0, nblk, block, 0)
        pltpu.sync_copy(tab.at[pl.ds(0, NPB * ch)],
                        out_hbm.at[pl.ds(lo * ch, NPB * ch)])

    f = pl.kernel(
        body,
        out_type=jax.ShapeDtypeStruct((NP * ch,), jnp.float32),
        mesh=_mesh(),
        compiler_params=pltpu.CompilerParams(use_tc_tiling_on_sc=False, needs_layout_passes=False),
        scratch_types=[
            pltpu.VMEM((FB,), jnp.int32),
            pltpu.VMEM((FB,), jnp.int32),
            pltpu.VMEM((TROWS * ch,), jnp.float32),
            pltpu.VMEM((L, ch), jnp.float32),
            pltpu.VMEM((L, ch), jnp.float32),
            pltpu.VMEM((L,), jnp.int32),
            pltpu.SemaphoreType.DMA,
            pltpu.SemaphoreType.DMA,
        ],
    )
    ids, rows, counts = bkt
    return f(h2, ids, rows, counts).reshape(NP, ch)


@functools.partial(jax.jit, static_argnames=("relu", "exact"))
def _tc_matmul(X, W, b, *, relu, exact=False):
    M, K = X.shape
    Nc = W.shape[1]
    BM = 256
    BN = min(Nc, 512)
    prec = jax.lax.Precision.HIGHEST if exact else None

    def body(x_ref, w_ref, b_ref, o_ref):
        y = jnp.dot(x_ref[...], w_ref[...],
                    preferred_element_type=jnp.float32, precision=prec) + b_ref[...]
        if relu:
            y = jnp.maximum(y, 0.0)
        o_ref[...] = y

    return pl.pallas_call(
        body,
        grid=(M // BM, Nc // BN),
        in_specs=[
            pl.BlockSpec((BM, K), lambda i, j: (i, 0)),
            pl.BlockSpec((K, BN), lambda i, j: (0, j)),
            pl.BlockSpec((1, BN), lambda i, j: (0, j)),
        ],
        out_specs=pl.BlockSpec((BM, BN), lambda i, j: (i, j)),
        out_shape=jax.ShapeDtypeStruct((M, Nc), jnp.float32),
    )(X, W, b)


@jax.jit
def _tc_edge_mlp(xi, dx, dpos, w1a, w1b, pw16, pb, wpf, w2, b1, b2):
    cin = xi.shape[1]
    ch = w2.shape[0]
    BM = 256

    def body(xi_ref, dx_ref, d_ref, w1a_ref, w1b_ref, pw_ref, pb_ref,
             wpf_ref, w2_ref, b1_ref, b2_ref, o_ref):
        dot = functools.partial(jnp.dot, preferred_element_type=jnp.float32)
        pf = jnp.maximum(dot(d_ref[...], pw_ref[...]) + pb_ref[...], 0.0)
        m1 = (dot(xi_ref[...], w1a_ref[...]) + dot(dx_ref[...], w1b_ref[...])
              + dot(pf, wpf_ref[...]) + b1_ref[...])
        h1 = jnp.maximum(m1, 0.0)
        o_ref[...] = jnp.maximum(
            dot(h1, w2_ref[...]) + b2_ref[...], 0.0)

    return pl.pallas_call(
        body,
        grid=(E // BM,),
        in_specs=[
            pl.BlockSpec((BM, cin), lambda i: (i, 0)),
            pl.BlockSpec((BM, cin), lambda i: (i, 0)),
            pl.BlockSpec((BM, 16), lambda i: (i, 0)),
            pl.BlockSpec((cin, ch), lambda i: (0, 0)),
            pl.BlockSpec((cin, ch), lambda i: (0, 0)),
            pl.BlockSpec((16, 16), lambda i: (0, 0)),
            pl.BlockSpec((1, 16), lambda i: (0, 0)),
            pl.BlockSpec((16, ch), lambda i: (0, 0)),
            pl.BlockSpec((ch, ch), lambda i: (0, 0)),
            pl.BlockSpec((1, ch), lambda i: (0, 0)),
            pl.BlockSpec((1, ch), lambda i: (0, 0)),
        ],
        out_specs=pl.BlockSpec((BM, ch), lambda i: (i, 0)),
        out_shape=jax.ShapeDtypeStruct((E, ch), jnp.float32),
    )(xi, dx, dpos, w1a, w1b, pw16, pb, wpf, w2, b1, b2)


@jax.jit
def _tc_pool(x4, oh):
    BM = 256
    D = x4.shape[1]

    def body(x_ref, oh_ref, o_ref):
        @pl.when(pl.program_id(0) == 0)
        def _():
            o_ref[...] = jnp.zeros_like(o_ref)
        x = x_ref[...]
        for g in range(NG):
            cand = x * oh_ref[:, g:g + 1]
            o_ref[g:g + 1, :] = jnp.maximum(
                o_ref[g:g + 1, :], jnp.max(cand, axis=0, keepdims=True))

    return pl.pallas_call(
        body,
        grid=(NP // BM,),
        in_specs=[
            pl.BlockSpec((BM, D), lambda i: (i, 0)),
            pl.BlockSpec((BM, NG), lambda i: (i, 0)),
        ],
        out_specs=pl.BlockSpec((NG, D), lambda i: (0, 0)),
        out_shape=jax.ShapeDtypeStruct((NG, D), jnp.float32),
    )(x4, oh)


def _pad_rows(a, rows):
    return jnp.pad(a, ((0, rows - a.shape[0]), (0, 0)))


def _conv(x, ch, p, gath, bkt):
    cin = x.shape[1]
    (w1, b1), (w2, b2) = p["msg"]
    pw, pb = p["pos"]
    w_xi, w_dx, w_pf = w1[:cin], w1[cin:2 * cin], w1[2 * cin:]
    pw16 = jnp.pad(pw, ((0, 13), (0, 0)))
    xi, dx, dpos = gath
    h2 = _tc_edge_mlp(xi, dx, dpos, w_xi, w_dx, pw16, pb[None, :], w_pf,
                      w2, b1[None, :], b2[None, :])
    return _sc_scatter_max(h2, bkt, ch=ch)


def _gcu(x, cout, p, ei_t, ei_g, posp, bkt_t, bkt_g):
    cin = x.shape[1]
    g_t = _sc_gather_combine(x, posp, ei_t[0], ei_t[1], cin=cin)
    g_g = _sc_gather_combine(x, posp, ei_g[0], ei_g[1], cin=cin)
    a = _conv(x, cout // 2, p["tpl"], g_t, bkt_t)
    b = _conv(x, cout // 2, p["geo"], g_g, bkt_g)
    wm, bm = p["mlp"][0]
    return _tc_matmul(jnp.concatenate([a, b], axis=1), wm, bm[None, :],
                      relu=True)


def kernel(pos, feature, tpl_edge_index, geo_edge_index, batch, params):
    posp = _pad_rows(jnp.pad(pos, ((0, 0), (0, 13))), NP)
    xf = _pad_rows(feature, NP)
    batchp = jnp.pad(batch, (0, NP - N), constant_values=NG)
    oh = (batchp[:, None] == jnp.arange(NG, dtype=jnp.int32)[None, :]
          ).astype(jnp.float32)

    bkt_t = _sc_bucket_build(tpl_edge_index[1])
    bkt_g = _sc_bucket_build(geo_edge_index[1])
    x1 = _gcu(xf, 64, params["gcu1"], tpl_edge_index, geo_edge_index, posp,
              bkt_t, bkt_g)
    x2 = _gcu(x1, 256, params["gcu2"], tpl_edge_index, geo_edge_index, posp,
              bkt_t, bkt_g)
    x3 = _gcu(x2, 512, params["gcu3"], tpl_edge_index, geo_edge_index, posp,
              bkt_t, bkt_g)

    wg, bg = params["glb"][0]
    x4 = _tc_matmul(jnp.concatenate([x1, x2, x3], axis=1), wg, bg[None, :],
                    relu=True)
    xg = _tc_pool(x4, oh)
    xgb = _tc_matmul(oh, xg, jnp.zeros((1, xg.shape[1]), jnp.float32),
                     relu=False, exact=True)

    x5 = jnp.concatenate([xgb, posp[:, :3], xf, x1, x2, x3], axis=1)
    x5 = jnp.pad(x5, ((0, 0), (0, 2048 - x5.shape[1])))
    (wt1, bt1), (wt2, bt2) = params["trans"]
    wt1p = jnp.pad(wt1, ((0, 2048 - wt1.shape[0]), (0, 0)))
    t1 = _tc_matmul(x5, wt1p, bt1[None, :], relu=True)
    t2 = _tc_matmul(t1, wt2, bt2[None, :], relu=True)
    wo, bo = params["trans_out"]
    wop = jnp.pad(wo, ((0, 0), (0, 128 - wo.shape[1])))
    bop = jnp.pad(bo, (0, 128 - bo.shape[0]))
    o = _tc_matmul(t2, wop, bop[None, :], relu=False)
    return o[:N, :3]
